# baseline (speedup 1.0000x reference)
import functools

import jax
import jax.numpy as jnp
from jax import lax
from jax.experimental import pallas as pl
from jax.experimental.pallas import tpu as pltpu
from jax.experimental.pallas import tpu_sc as plsc

B = 16
Q = 256
T = 16384
TH = T // 2
CHUNK = 128
NCHUNK = TH // CHUNK
ZBLK = 1 << 20


def _zero_body(out_ref):
    out_ref[...] = jnp.zeros((ZBLK,), jnp.float32)


def _tc_zeros():
    return pl.pallas_call(
        _zero_body,
        grid=(B * Q * T // ZBLK,),
        out_specs=pl.BlockSpec((ZBLK,), lambda i: (i,)),
        out_shape=jax.ShapeDtypeStruct((B * Q * T,), jnp.float32),
    )()


def _sc_scatter_body(idx_hbm, out_ref, idx_v, idxs_v, ones_v, sem_s):
    b = lax.axis_index("s")
    half = lax.axis_index("c")
    t0 = half * TH
    base = b * (Q * T)

    pltpu.sync_copy(idx_hbm.at[b, pl.ds(t0, TH)], idx_v)

    def oinit(u, _):
        ones_v[pl.ds(u * 16, 16)] = jnp.full((16,), 1.0, jnp.float32)
        return 0

    lax.fori_loop(0, CHUNK // 16, oinit, 0)

    lane = lax.iota(jnp.int32, 16)

    def cchunk(j, _):
        def cvec(u, _):
            toff = t0 + j * CHUNK + u * 16
            tpart = base + ((toff >> 7) << 10) + (toff & 127)
            v = idx_v[pl.ds(j * CHUNK + u * 16, 16)]
            qpart = (v & 7) << 7
            idxs_v[j, pl.ds(u * 16, 16)] = qpart + tpart + lane
            return 0

        lax.fori_loop(0, CHUNK // 16, cvec, 0)
        return 0

    lax.fori_loop(0, NCHUNK, cchunk, 0)

    def sfire(j, _):
        pltpu.make_async_copy(ones_v, out_ref.at[idxs_v.at[j]], sem_s).start()
        return 0

    lax.fori_loop(0, NCHUNK, sfire, 0)

    def sdrain(j, _):
        pltpu.make_async_copy(ones_v, out_ref.at[idxs_v.at[0]], sem_s).wait()
        return 0

    lax.fori_loop(0, NCHUNK, sdrain, 0)


_sc_scatter = functools.partial(
    pl.kernel,
    mesh=plsc.VectorSubcoreMesh(core_axis_name="c", subcore_axis_name="s"),
    scratch_types=[
        pltpu.VMEM((TH,), jnp.int32),
        pltpu.VMEM((NCHUNK, CHUNK), jnp.int32),
        pltpu.VMEM((CHUNK,), jnp.float32),
        pltpu.SemaphoreType.DMA,
    ],
)(_sc_scatter_body)


def kernel(in_snd_slice, quant_onehot):
    del quant_onehot
    idx = in_snd_slice.astype(jnp.int32)
    out_ref = jax.new_ref(_tc_zeros())
    _sc_scatter(idx, out_ref)
    flat = jax.freeze(out_ref)
    return (
        flat.reshape(B, 32, 128, 8, 128)
        .transpose(0, 1, 3, 2, 4)
        .reshape(B, Q, T)
    )


# speedup vs baseline: 1.3702x; 1.3702x over previous
import functools

import jax
import jax.numpy as jnp
from jax import lax
from jax.experimental import pallas as pl
from jax.experimental.pallas import tpu as pltpu
from jax.experimental.pallas import tpu_sc as plsc

B = 16
Q = 256
T = 16384
TH = T // 2
CHUNK = 128
NCHUNK = TH // CHUNK
ZBLK = 1 << 20


def _zero_body(out_ref):
    out_ref[...] = jnp.zeros((ZBLK,), jnp.float32)


def _tc_zeros():
    return pl.pallas_call(
        _zero_body,
        grid=(B * Q * T // ZBLK,),
        out_specs=pl.BlockSpec((ZBLK,), lambda i: (i,)),
        out_shape=jax.ShapeDtypeStruct((B * Q * T,), jnp.float32),
    )()


def _sc_scatter_body(idx_hbm, out_ref, idx_v, idxs_v, ones_v, sem_s):
    b = lax.axis_index("s")
    half = lax.axis_index("c")
    t0 = half * TH
    base = b * (Q * T)

    pltpu.sync_copy(idx_hbm.at[b, pl.ds(t0, TH)], idx_v)

    def oinit(u, _):
        ones_v[pl.ds(u * 16, 16)] = jnp.full((16,), 1.0, jnp.float32)
        return 0

    lax.fori_loop(0, CHUNK // 16, oinit, 0)

    lane = lax.iota(jnp.int32, 16)

    def cchunk(j, _):
        def cvec(u, _):
            toff = t0 + j * CHUNK + u * 16
            tpart = base + ((toff >> 7) << 10) + (toff & 127)
            v = idx_v[pl.ds(j * CHUNK + u * 16, 16)]
            qpart = ((v >> 3) << 17) + ((v & 7) << 7)
            idxs_v[j, pl.ds(u * 16, 16)] = qpart + tpart + lane
            return 0

        lax.fori_loop(0, CHUNK // 16, cvec, 0)
        pltpu.make_async_copy(ones_v, out_ref.at[idxs_v.at[j]], sem_s).start()
        return 0

    lax.fori_loop(0, NCHUNK, cchunk, 0)

    def sdrain(j, _):
        pltpu.make_async_copy(ones_v, out_ref.at[idxs_v.at[0]], sem_s).wait()
        return 0

    lax.fori_loop(0, NCHUNK, sdrain, 0)


_sc_scatter = functools.partial(
    pl.kernel,
    mesh=plsc.VectorSubcoreMesh(core_axis_name="c", subcore_axis_name="s"),
    scratch_types=[
        pltpu.VMEM((TH,), jnp.int32),
        pltpu.VMEM((NCHUNK, CHUNK), jnp.int32),
        pltpu.VMEM((CHUNK,), jnp.float32),
        pltpu.SemaphoreType.DMA,
    ],
)(_sc_scatter_body)


def kernel(in_snd_slice, quant_onehot):
    del quant_onehot
    idx = in_snd_slice.astype(jnp.int32)
    out_ref = jax.new_ref(_tc_zeros())
    _sc_scatter(idx, out_ref)
    flat = jax.freeze(out_ref)
    return (
        flat.reshape(B, 32, 128, 8, 128)
        .transpose(0, 1, 3, 2, 4)
        .reshape(B, Q, T)
    )


# speedup vs baseline: 1.3712x; 1.0008x over previous
import functools

import jax
import jax.numpy as jnp
from jax import lax
from jax.experimental import pallas as pl
from jax.experimental.pallas import tpu as pltpu
from jax.experimental.pallas import tpu_sc as plsc

B = 16
Q = 256
T = 16384
TH = T // 2
CHUNK = 64
NCHUNK = TH // CHUNK
ZBLK = 1 << 20


def _zero_body(out_ref):
    out_ref[...] = jnp.zeros((ZBLK,), jnp.float32)


def _tc_zeros():
    return pl.pallas_call(
        _zero_body,
        grid=(B * Q * T // ZBLK,),
        out_specs=pl.BlockSpec((ZBLK,), lambda i: (i,)),
        out_shape=jax.ShapeDtypeStruct((B * Q * T,), jnp.float32),
    )()


def _sc_scatter_body(idx_hbm, out_ref, idx_v, idxs_v, ones_v, sem_s):
    b = lax.axis_index("s")
    half = lax.axis_index("c")
    t0 = half * TH
    base = b * (Q * T)

    pltpu.sync_copy(idx_hbm.at[b, pl.ds(t0, TH)], idx_v)

    def oinit(u, _):
        ones_v[pl.ds(u * 16, 16)] = jnp.full((16,), 1.0, jnp.float32)
        return 0

    lax.fori_loop(0, CHUNK // 16, oinit, 0)

    lane = lax.iota(jnp.int32, 16)

    def cchunk(j, _):
        def cvec(u, _):
            toff = t0 + j * CHUNK + u * 16
            tpart = base + ((toff >> 7) << 10) + (toff & 127)
            v = idx_v[pl.ds(j * CHUNK + u * 16, 16)]
            qpart = ((v >> 3) << 17) + ((v & 7) << 7)
            idxs_v[j, pl.ds(u * 16, 16)] = qpart + tpart + lane
            return 0

        lax.fori_loop(0, CHUNK // 16, cvec, 0)
        pltpu.make_async_copy(ones_v, out_ref.at[idxs_v.at[j]], sem_s).start()
        return 0

    lax.fori_loop(0, NCHUNK, cchunk, 0)

    def sdrain(j, _):
        pltpu.make_async_copy(ones_v, out_ref.at[idxs_v.at[0]], sem_s).wait()
        return 0

    lax.fori_loop(0, NCHUNK, sdrain, 0)


_sc_scatter = functools.partial(
    pl.kernel,
    mesh=plsc.VectorSubcoreMesh(core_axis_name="c", subcore_axis_name="s"),
    scratch_types=[
        pltpu.VMEM((TH,), jnp.int32),
        pltpu.VMEM((NCHUNK, CHUNK), jnp.int32),
        pltpu.VMEM((CHUNK,), jnp.float32),
        pltpu.SemaphoreType.DMA,
    ],
)(_sc_scatter_body)


def kernel(in_snd_slice, quant_onehot):
    del quant_onehot
    idx = in_snd_slice.astype(jnp.int32)
    out_ref = jax.new_ref(_tc_zeros())
    _sc_scatter(idx, out_ref)
    flat = jax.freeze(out_ref)
    return (
        flat.reshape(B, 32, 128, 8, 128)
        .transpose(0, 1, 3, 2, 4)
        .reshape(B, Q, T)
    )
